# baseline (speedup 1.0000x reference)
import jax
import jax.numpy as jnp
from jax import lax
from jax.experimental import pallas as pl
from jax.experimental.pallas import tpu as pltpu
from jax.experimental.pallas import tpu_sc as plsc

B = 8
N_FINE = 40962
D = 128
K = 7
N_COARSE = 10242

NW = 32
PER_W = 320
N_MAIN = NW * PER_W
N_TAIL = N_COARSE - N_MAIN
CHUNK = 2
NCHUNK = PER_W // CHUNK
IDX_C = CHUNK * K
IDX_P = 16
NBUF = 5
DBLK = D // 16


def _tree_max7(rows):
    t0 = jnp.maximum(rows[0], rows[1])
    t1 = jnp.maximum(rows[2], rows[3])
    t2 = jnp.maximum(rows[4], rows[5])
    return jnp.maximum(jnp.maximum(t0, t1), jnp.maximum(t2, rows[6]))


def _pool_body(x_hbm, idxm_hbm, idxt_hbm, out_hbm,
               idx_v, gbuf0, gbuf1, gbuf2, gbuf3, gbuf4,
               obuf0, obuf1, obuf2, obuf3, obuf4, idxt_raw,
               semG0, semG1, semG2, semG3, semG4, semS, semT):
    gbufs = (gbuf0, gbuf1, gbuf2, gbuf3, gbuf4)
    obufs = (obuf0, obuf1, obuf2, obuf3, obuf4)
    semGs = (semG0, semG1, semG2, semG3, semG4)
    wid = lax.axis_index("s") * 2 + lax.axis_index("c")
    base_c = wid * PER_W
    pltpu.sync_copy(idxm_hbm.at[pl.ds(wid * (NCHUNK * IDX_P), NCHUNK * IDX_P)],
                    idx_v)

    def fire(p, j):
        idx_slice = idx_v.at[pl.ds(j * IDX_P, IDX_C)]
        pltpu.async_copy(x_hbm.at[idx_slice], gbufs[p], semGs[p])

    def wait_gather(p):
        idx_slice = idx_v.at[pl.ds(0, IDX_C)]
        pltpu.make_async_copy(x_hbm.at[idx_slice], gbufs[p], semGs[p]).wait()

    def wait_store(p):
        pltpu.make_async_copy(obufs[p], out_hbm.at[pl.ds(0, CHUNK)],
                              semS).wait()

    def compute(p):
        gbuf = gbufs[p]
        obuf = obufs[p]

        @plsc.parallel_loop(0, CHUNK * B, 1, unroll=4)
        def _(i):
            c = lax.shift_right_logical(i, 3)
            b = lax.bitwise_and(i, 7)
            for dblk in range(DBLK):
                o = pl.ds(dblk * 16, 16)
                m = _tree_max7([gbuf[K * c + k2, b, o] for k2 in range(K)])
                obuf[c, b, o] = m

    for p in range(NBUF):
        fire(p, p)

    def quad_body(j4, carry):
        @pl.when(j4 >= 1)
        def _():
            for p in range(NBUF):
                wait_store(p)

        for p in range(NBUF):
            c = NBUF * j4 + p
            wait_gather(p)
            compute(p)
            pltpu.async_copy(obufs[p],
                             out_hbm.at[pl.ds(base_c + c * CHUNK, CHUNK)],
                             semS)

            @pl.when(c + NBUF < NCHUNK)
            def _():
                fire(p, c + NBUF)

        return carry

    lax.fori_loop(0, NCHUNK // NBUF, quad_body, 0)
    for p in range(NBUF):
        wait_store(p)

    @pl.when(wid == NW - 1)
    def _():
        pltpu.sync_copy(idxt_hbm, idxt_raw)
        idx_slice = idxt_raw.at[pl.ds(0, IDX_C)]
        pltpu.async_copy(x_hbm.at[idx_slice], gbuf0, semT).wait()
        for c in range(N_TAIL):
            for b in range(B):
                for dblk in range(DBLK):
                    o = pl.ds(dblk * 16, 16)
                    m = _tree_max7([gbuf0[K * c + k2, b, o]
                                    for k2 in range(K)])
                    obuf0[c, b, o] = m
        pltpu.sync_copy(obuf0, out_hbm.at[pl.ds(N_MAIN, N_TAIL)])


def kernel(x, pool_idx):
    idx = pool_idx.astype(jnp.int32)
    idx_main = jnp.pad(idx[:N_MAIN].reshape(NW * NCHUNK, IDX_C),
                       ((0, 0), (0, IDX_P - IDX_C))).reshape(-1)
    idx_tail = jnp.pad(idx[N_MAIN:].reshape(N_TAIL * K), (0, 16 - N_TAIL * K))
    x_t = jnp.transpose(x, (1, 0, 2))

    mesh = plsc.VectorSubcoreMesh(core_axis_name="c", subcore_axis_name="s")
    f = pl.kernel(
        _pool_body,
        mesh=mesh,
        out_type=jax.ShapeDtypeStruct((N_COARSE, B, D), jnp.float32),
        scratch_types=[
            pltpu.VMEM((NCHUNK * IDX_P,), jnp.int32),
            pltpu.VMEM((IDX_C, B, D), jnp.float32),
            pltpu.VMEM((IDX_C, B, D), jnp.float32),
            pltpu.VMEM((IDX_C, B, D), jnp.float32),
            pltpu.VMEM((IDX_C, B, D), jnp.float32),
            pltpu.VMEM((IDX_C, B, D), jnp.float32),
            pltpu.VMEM((CHUNK, B, D), jnp.float32),
            pltpu.VMEM((CHUNK, B, D), jnp.float32),
            pltpu.VMEM((CHUNK, B, D), jnp.float32),
            pltpu.VMEM((CHUNK, B, D), jnp.float32),
            pltpu.VMEM((CHUNK, B, D), jnp.float32),
            pltpu.VMEM((16,), jnp.int32),
            pltpu.SemaphoreType.DMA,
            pltpu.SemaphoreType.DMA,
            pltpu.SemaphoreType.DMA,
            pltpu.SemaphoreType.DMA,
            pltpu.SemaphoreType.DMA,
            pltpu.SemaphoreType.DMA,
            pltpu.SemaphoreType.DMA,
        ],
    )
    out_t = f(x_t, idx_main, idx_tail)
    return jnp.transpose(out_t, (1, 0, 2))


# speedup vs baseline: 1.6676x; 1.6676x over previous
import jax
import jax.numpy as jnp
from jax import lax
from jax.experimental import pallas as pl
from jax.experimental.pallas import tpu as pltpu
from jax.experimental.pallas import tpu_sc as plsc

B = 8
N_FINE = 40962
D = 128
K = 7
N_COARSE = 10242

NW = 32
CHUNK = 2
NCHUNKS = N_COARSE // CHUNK
PER_W = NCHUNKS // NW
IDX_C = CHUNK * K
IDX_P = 16
NBUF = 4
NQ = PER_W // NBUF + 1
DBLK = D // 16
STAGE_W = (PER_W + 1) * IDX_P


def _tree_max7(rows):
    t0 = jnp.maximum(rows[0], rows[1])
    t1 = jnp.maximum(rows[2], rows[3])
    t2 = jnp.maximum(rows[4], rows[5])
    return jnp.maximum(jnp.maximum(t0, t1), jnp.maximum(t2, rows[6]))


def _pool_body(x_hbm, idxm_hbm, out_hbm,
               idx_v, gbuf0, gbuf1, gbuf2, gbuf3,
               obuf0, obuf1, obuf2, obuf3,
               semG0, semG1, semG2, semG3, semS):
    gbufs = (gbuf0, gbuf1, gbuf2, gbuf3)
    obufs = (obuf0, obuf1, obuf2, obuf3)
    semGs = (semG0, semG1, semG2, semG3)
    wid = lax.axis_index("s") * 2 + lax.axis_index("c")
    base_chunk = wid * PER_W
    nc = jnp.where(wid == NW - 1, PER_W + 1, PER_W)
    pltpu.sync_copy(idxm_hbm.at[pl.ds(wid * (PER_W * IDX_P), STAGE_W)], idx_v)

    def fire(p, j):
        idx_slice = idx_v.at[pl.ds(j * IDX_P, IDX_C)]
        pltpu.async_copy(x_hbm.at[idx_slice], gbufs[p], semGs[p])

    def wait_gather(p):
        idx_slice = idx_v.at[pl.ds(0, IDX_C)]
        pltpu.make_async_copy(x_hbm.at[idx_slice], gbufs[p], semGs[p]).wait()

    def wait_store(p):
        pltpu.make_async_copy(obufs[p], out_hbm.at[pl.ds(0, CHUNK)],
                              semS).wait()

    def compute(p):
        gbuf = gbufs[p]
        obuf = obufs[p]

        @plsc.parallel_loop(0, CHUNK * B, 1, unroll=2)
        def _(i):
            c = lax.shift_right_logical(i, 3)
            b = lax.bitwise_and(i, 7)
            for dblk in range(DBLK):
                o = pl.ds(dblk * 16, 16)
                m = _tree_max7([gbuf[K * c + k2, b, o] for k2 in range(K)])
                obuf[c, b, o] = m

    for p in range(NBUF):
        fire(p, p)

    def ring_body(j4, carry):
        @pl.when(j4 >= 1)
        def _():
            for p in range(NBUF):
                wait_store(p)

        for p in range(NBUF):
            c = NBUF * j4 + p

            @pl.when(c < nc)
            def _():
                wait_gather(p)
                compute(p)
                pltpu.async_copy(
                    obufs[p],
                    out_hbm.at[pl.ds((base_chunk + c) * CHUNK, CHUNK)],
                    semS)

            @pl.when(c + NBUF < nc)
            def _():
                fire(p, c + NBUF)

        return carry

    lax.fori_loop(0, NQ, ring_body, 0)
    for p in range(NBUF):
        @pl.when(NBUF * (NQ - 1) + p < nc)
        def _():
            wait_store(p)


def kernel(x, pool_idx):
    idx = pool_idx.astype(jnp.int32)
    idx_main = jnp.pad(idx.reshape(NCHUNKS, IDX_C),
                       ((0, 0), (0, IDX_P - IDX_C))).reshape(-1)
    x_t = jnp.transpose(x, (1, 0, 2))

    mesh = plsc.VectorSubcoreMesh(core_axis_name="c", subcore_axis_name="s")
    f = pl.kernel(
        _pool_body,
        mesh=mesh,
        out_type=jax.ShapeDtypeStruct((N_COARSE, B, D), jnp.float32),
        scratch_types=[
            pltpu.VMEM((STAGE_W,), jnp.int32),
            pltpu.VMEM((IDX_C, B, D), jnp.float32),
            pltpu.VMEM((IDX_C, B, D), jnp.float32),
            pltpu.VMEM((IDX_C, B, D), jnp.float32),
            pltpu.VMEM((IDX_C, B, D), jnp.float32),
            pltpu.VMEM((CHUNK, B, D), jnp.float32),
            pltpu.VMEM((CHUNK, B, D), jnp.float32),
            pltpu.VMEM((CHUNK, B, D), jnp.float32),
            pltpu.VMEM((CHUNK, B, D), jnp.float32),
            pltpu.SemaphoreType.DMA,
            pltpu.SemaphoreType.DMA,
            pltpu.SemaphoreType.DMA,
            pltpu.SemaphoreType.DMA,
            pltpu.SemaphoreType.DMA,
        ],
    )
    out_t = f(x_t, idx_main)
    return jnp.transpose(out_t, (1, 0, 2))
